# Initial kernel scaffold; baseline (speedup 1.0000x reference)
import jax
import jax.numpy as jnp
from jax.experimental import pallas as pl


def kernel(node_feats, edge_weights, node_weights, suppl1, suppl2, suppl3, AR1, AR2, AR3, AR4, W1, W2, W3, gn1_gamma, gn1_beta, gn1_alpha, gn2_gamma, gn2_beta, gn2_alpha, gn3_gamma, gn3_beta, gn3_alpha, r1_phi_w, r1_phi_b, r1_rho_w, r1_rho_b, r2_phi_w, r2_phi_b, r2_rho_w, r2_rho_b, r3_phi_w, r3_phi_b, r3_rho_w, r3_rho_b, edge_index):
    raise NotImplementedError("write your pallas kernel here")


# speedup vs baseline: 5.0393x; 5.0393x over previous
import jax
import jax.numpy as jnp
from jax import lax
from jax.experimental import pallas as pl
from jax.experimental.pallas import tpu as pltpu
from jax.experimental.pallas import tpu_sc as plsc

_N = 10000
_E = 320000
_F = 128
_R = 64
_NPAD = 10240
_NC = 2
_NS = 16
_NW = _NC * _NS
_ER = _E // 128
_RPW = _ER // _NW
_REM = _ER - _RPW * _NW
_WR = 2
_NWIN = _RPW // _WR
_BN = 2000


def _leaky(x):
    return jnp.where(x >= 0, x, 0.01 * x)


def _deg_body(src1, dst1, out_hbm, ho, hi, sbuf, dbuf):
    c = lax.axis_index("c")
    s = lax.axis_index("s")
    wid = c * _NS + s

    def _z(i, _):
        ho[pl.ds(i * 16, 16)] = jnp.zeros((16,), jnp.float32)
        hi[pl.ds(i * 16, 16)] = jnp.zeros((16,), jnp.float32)
        return 0

    lax.fori_loop(0, _NPAD // 16, _z, 0)

    base = (wid * _RPW + jnp.minimum(wid, _REM)) * 128
    ne = _RPW * 128
    pltpu.sync_copy(src1.at[pl.ds(base, ne)], sbuf.at[pl.ds(0, ne)])
    pltpu.sync_copy(dst1.at[pl.ds(base, ne)], dbuf.at[pl.ds(0, ne)])

    @pl.when(wid < _REM)
    def _():
        pltpu.sync_copy(src1.at[pl.ds(base + ne, 128)],
                        sbuf.at[pl.ds(ne, 128)])
        pltpu.sync_copy(dst1.at[pl.ds(base + ne, 128)],
                        dbuf.at[pl.ds(ne, 128)])

    ones = jnp.full((16,), 1.0, jnp.float32)

    def _sc(i, _):
        iv = sbuf[pl.ds(i * 16, 16)]
        plsc.addupdate_scatter(ho, [iv], ones)
        jv = dbuf[pl.ds(i * 16, 16)]
        plsc.addupdate_scatter(hi, [jv], ones)
        return 0

    lax.fori_loop(0, ne // 16, _sc, 0)

    @pl.when(wid < _REM)
    def _():
        lax.fori_loop(ne // 16, ne // 16 + 8, _sc, 0)

    pltpu.sync_copy(ho, out_hbm.at[pl.ds(wid * _NPAD, _NPAD)])
    pltpu.sync_copy(hi, out_hbm.at[pl.ds((_NW + wid) * _NPAD, _NPAD)])


def _prop_body(h_hbm, src1, dst1, ew1, sp1, out_hbm,
               acc, sbufs, dbufs, ewb, spb, rows, sem):
    c = lax.axis_index("c")
    s = lax.axis_index("s")
    wid = c * _NS + s

    def _z(i, _):
        r = i // 8
        k = i % 8
        rows[r, pl.ds(k * 16, 16)] = jnp.zeros((16,), jnp.float32)
        return 0

    lax.fori_loop(0, _WR * 128 * 8, _z, 0)
    pltpu.sync_copy(rows.at[pl.ds(0, 256)], acc.at[pl.ds(s * 624, 256)])
    pltpu.sync_copy(rows.at[pl.ds(0, 256)], acc.at[pl.ds(s * 624 + 256, 256)])
    pltpu.sync_copy(rows.at[pl.ds(0, 112)], acc.at[pl.ds(s * 624 + 512, 112)])

    @pl.when(s == _NS - 1)
    def _():
        pltpu.sync_copy(rows.at[pl.ds(0, 16)], acc.at[pl.ds(9984, 16)])

    plsc.subcore_barrier()

    ebase = (wid * _RPW + jnp.minimum(wid, _REM)) * 128

    def _do_window(e0, nch):
        for j in range(nch):
            pltpu.sync_copy(src1.at[pl.ds(e0 + j * 128, 128)], sbufs[j])
            pltpu.sync_copy(dst1.at[pl.ds(e0 + j * 128, 128)], dbufs[j])
        pltpu.sync_copy(ew1.at[pl.ds(e0, nch * 128)],
                        ewb.at[pl.ds(0, nch * 128)])
        pltpu.sync_copy(sp1.at[pl.ds(e0, nch * 128)],
                        spb.at[pl.ds(0, nch * 128)])
        descs = [
            pltpu.async_copy(h_hbm.at[sbufs[j]],
                             rows.at[pl.ds(j * 128, 128)], sem)
            for j in range(nch)
        ]
        for d in descs:
            d.wait()
        for j in range(nch):
            def _grp(g, _, j=j):
                wv = (ewb[pl.ds(j * 128 + g * 16, 16)]
                      * spb[pl.ds(j * 128 + g * 16, 16)])
                for q in range(16):
                    w = wv[q]
                    r = j * 128 + g * 16 + q
                    for f in range(8):
                        sl = pl.ds(f * 16, 16)
                        rows[r, sl] = rows[r, sl] * w
                return 0

            lax.fori_loop(0, 8, _grp, 0)
        for j in range(nch):
            pltpu.sync_copy(rows.at[pl.ds(j * 128, 128)],
                            acc.at[dbufs[j]], add=True)

    def _main(wv, _):
        _do_window(ebase + wv * (_WR * 128), _WR)
        return 0

    lax.fori_loop(0, _NWIN, _main, 0)

    @pl.when(wid < _REM)
    def _():
        _do_window(ebase + _RPW * 128, 1)

    plsc.subcore_barrier()
    pltpu.sync_copy(acc.at[pl.ds(s * 624, 624)],
                    out_hbm.at[c, pl.ds(s * 624, 624)])

    @pl.when(s == _NS - 1)
    def _():
        pltpu.sync_copy(acc.at[pl.ds(9984, 16)],
                        out_hbm.at[c, pl.ds(9984, 16)])


_sc_kernels = {}


def _get_sc_kernels():
    if "deg" not in _sc_kernels:
        mesh = plsc.VectorSubcoreMesh(
            core_axis_name="c", subcore_axis_name="s",
            num_cores=_NC, num_subcores=_NS)
        _sc_kernels["deg"] = pl.kernel(
            _deg_body,
            out_type=jax.ShapeDtypeStruct((2 * _NW * _NPAD,), jnp.float32),
            mesh=mesh,
            compiler_params=pltpu.CompilerParams(needs_layout_passes=False),
            scratch_types=[
                pltpu.VMEM((_NPAD,), jnp.float32),
                pltpu.VMEM((_NPAD,), jnp.float32),
                pltpu.VMEM(((_RPW + 1) * 128,), jnp.int32),
                pltpu.VMEM(((_RPW + 1) * 128,), jnp.int32),
            ],
        )
        _sc_kernels["prop"] = pl.kernel(
            _prop_body,
            out_type=jax.ShapeDtypeStruct((2, _N, _F), jnp.float32),
            mesh=mesh,
            scratch_types=[
                pltpu.VMEM_SHARED((_N, _F), jnp.float32),
                [pltpu.VMEM((128,), jnp.int32) for _ in range(_WR)],
                [pltpu.VMEM((128,), jnp.int32) for _ in range(_WR)],
                pltpu.VMEM((_WR * 128,), jnp.float32),
                pltpu.VMEM((_WR * 128,), jnp.float32),
                pltpu.VMEM((_WR * 128, 128), jnp.float32),
                pltpu.SemaphoreType.DMA,
            ],
        )
    return _sc_kernels


def _deg_call(srcr, dstr):
    return _get_sc_kernels()["deg"](srcr, dstr)


def _prop_call(hm, srcr, dstr, ewr, spr):
    return _get_sc_kernels()["prop"](hm, srcr, dstr, ewr, spr)


def _prep_body(h_ref, o_ref):
    x = h_ref[...]
    dego = jnp.sum(x[0:_NW], axis=0, keepdims=True)
    degi = jnp.sum(x[_NW:2 * _NW], axis=0, keepdims=True)
    deg = jnp.concatenate([dego, degi], axis=0)
    o_ref[...] = jax.lax.rsqrt(jnp.maximum(deg, 1.0))


def _a_body(x_ref, d_ref, nw_ref, ar_ref, w_ref, h_ref, ws_ref):
    i = pl.program_id(0)
    x = x_ref[...]
    h_ref[...] = jnp.dot(x * d_ref[...], w_ref[...],
                         preferred_element_type=jnp.float32)
    wm = jnp.sum(x * (nw_ref[...] * ar_ref[...]), axis=0, keepdims=True)

    @pl.when(i == 0)
    def _():
        ws_ref[...] = wm

    @pl.when(i != 0)
    def _():
        ws_ref[...] += wm


def _b1_body(p_ref, d_ref, agg_ref, st_ref):
    i = pl.program_id(0)
    p = p_ref[...]
    sgg = (p[0] + p[1]) * d_ref[...]
    agg_ref[...] = sgg
    st = jnp.concatenate(
        [jnp.sum(sgg, axis=0, keepdims=True),
         jnp.sum(sgg * sgg, axis=0, keepdims=True)], axis=0)

    @pl.when(i == 0)
    def _():
        st_ref[...] = st

    @pl.when(i != 0)
    def _():
        st_ref[...] += st


def _b2_body(agg_ref, st_ref, g_ref, b_ref, al_ref, nw_ref, ar_ref,
             pw_ref, pb_ref, h_ref, pp_ref, ws_ref):
    i = pl.program_id(0)
    st = st_ref[...]
    m = st[0:1] / _N
    msq = st[1:2] / _N
    al = al_ref[...]
    var = msq - (2.0 * al - al * al) * (m * m)
    stdv = jnp.sqrt(var + 1e-5)
    a = agg_ref[...]
    xn = g_ref[...] * (a - al * m) / stdv + b_ref[...]
    hl = _leaky(xn)
    h_ref[...] = hl
    phi = _leaky(jnp.dot(hl, pw_ref[...],
                         preferred_element_type=jnp.float32) + pb_ref[...])
    pp = jnp.sum(phi, axis=0, keepdims=True)
    wm = jnp.sum(hl * (nw_ref[...] * ar_ref[...]), axis=0, keepdims=True)

    @pl.when(i == 0)
    def _():
        pp_ref[...] = pp
        ws_ref[...] = wm

    @pl.when(i != 0)
    def _():
        pp_ref[...] += pp
        ws_ref[...] += wm


def _f_body(wm0, wm1, wm2, wm3, pp1, pp2, pp3,
            rw1, rw2, rw3, rb1, rb2, rb3, out_ref):
    rs = []
    for pp, rw, rb in ((pp1, rw1, rb1), (pp2, rw2, rb2), (pp3, rw3, rb3)):
        rs.append(_leaky(jnp.dot(pp[...] / _N, rw[...],
                                 preferred_element_type=jnp.float32) + rb[...]))
    out_ref[...] = _leaky(jnp.concatenate(
        [wm0[...] / _N, rs[0], wm1[...] / _N, rs[1],
         wm2[...] / _N, rs[2], wm3[...] / _N], axis=1))


def _tc_prep(hists):
    return pl.pallas_call(
        _prep_body,
        out_shape=jax.ShapeDtypeStruct((2, _NPAD), jnp.float32),
    )(hists)


def _tc_a(x, dvo, nwc, arc, w):
    g = _N // _BN
    return pl.pallas_call(
        _a_body,
        grid=(g,),
        in_specs=[
            pl.BlockSpec((_BN, _F), lambda i: (i, 0)),
            pl.BlockSpec((_BN, 1), lambda i: (i, 0)),
            pl.BlockSpec((_BN, 1), lambda i: (i, 0)),
            pl.BlockSpec((_BN, 1), lambda i: (i, 0)),
            pl.BlockSpec((_F, _F), lambda i: (0, 0)),
        ],
        out_specs=[
            pl.BlockSpec((_BN, _F), lambda i: (i, 0)),
            pl.BlockSpec((1, _F), lambda i: (0, 0)),
        ],
        out_shape=[
            jax.ShapeDtypeStruct((_N, _F), jnp.float32),
            jax.ShapeDtypeStruct((1, _F), jnp.float32),
        ],
    )(x, dvo, nwc, arc, w)


def _tc_b1(p, dvi):
    g = _N // _BN
    return pl.pallas_call(
        _b1_body,
        grid=(g,),
        in_specs=[
            pl.BlockSpec((2, _BN, _F), lambda i: (0, i, 0)),
            pl.BlockSpec((_BN, 1), lambda i: (i, 0)),
        ],
        out_specs=[
            pl.BlockSpec((_BN, _F), lambda i: (i, 0)),
            pl.BlockSpec((2, _F), lambda i: (0, 0)),
        ],
        out_shape=[
            jax.ShapeDtypeStruct((_N, _F), jnp.float32),
            jax.ShapeDtypeStruct((2, _F), jnp.float32),
        ],
    )(p, dvi)


def _tc_b2(agg, st, g2, b2, a2, nwc, arc, pw, pb):
    g = _N // _BN
    return pl.pallas_call(
        _b2_body,
        grid=(g,),
        in_specs=[
            pl.BlockSpec((_BN, _F), lambda i: (i, 0)),
            pl.BlockSpec((2, _F), lambda i: (0, 0)),
            pl.BlockSpec((1, _F), lambda i: (0, 0)),
            pl.BlockSpec((1, _F), lambda i: (0, 0)),
            pl.BlockSpec((1, _F), lambda i: (0, 0)),
            pl.BlockSpec((_BN, 1), lambda i: (i, 0)),
            pl.BlockSpec((_BN, 1), lambda i: (i, 0)),
            pl.BlockSpec((_F, _R), lambda i: (0, 0)),
            pl.BlockSpec((1, _R), lambda i: (0, 0)),
        ],
        out_specs=[
            pl.BlockSpec((_BN, _F), lambda i: (i, 0)),
            pl.BlockSpec((1, _R), lambda i: (0, 0)),
            pl.BlockSpec((1, _F), lambda i: (0, 0)),
        ],
        out_shape=[
            jax.ShapeDtypeStruct((_N, _F), jnp.float32),
            jax.ShapeDtypeStruct((1, _R), jnp.float32),
            jax.ShapeDtypeStruct((1, _F), jnp.float32),
        ],
    )(agg, st, g2, b2, a2, nwc, arc, pw, pb)


def _tc_final(wm0, wm1, wm2, wm3, pp1, pp2, pp3, rws, rbs):
    return pl.pallas_call(
        _f_body,
        out_shape=jax.ShapeDtypeStruct((1, 704), jnp.float32),
    )(wm0, wm1, wm2, wm3, pp1, pp2, pp3,
      rws[0], rws[1], rws[2], rbs[0], rbs[1], rbs[2])


def kernel(node_feats, edge_weights, node_weights, suppl1, suppl2, suppl3,
           AR1, AR2, AR3, AR4, W1, W2, W3,
           gn1_gamma, gn1_beta, gn1_alpha,
           gn2_gamma, gn2_beta, gn2_alpha,
           gn3_gamma, gn3_beta, gn3_alpha,
           r1_phi_w, r1_phi_b, r1_rho_w, r1_rho_b,
           r2_phi_w, r2_phi_b, r2_rho_w, r2_rho_b,
           r3_phi_w, r3_phi_b, r3_rho_w, r3_rho_b, edge_index):
    srcr = edge_index[0]
    dstr = edge_index[1]
    ewr = edge_weights
    spr = (suppl1, suppl2, suppl3)

    hists = _deg_call(srcr, dstr).reshape(2 * _NW, _NPAD)
    dinv2 = _tc_prep(hists)
    dvo = dinv2[0, :_N].reshape(_N, 1)
    dvi = dinv2[1, :_N].reshape(_N, 1)
    nwc = node_weights.reshape(_N, 1)
    ars = (AR1.reshape(_N, 1), AR2.reshape(_N, 1),
           AR3.reshape(_N, 1), AR4.reshape(_N, 1))
    gns = ((gn1_gamma, gn1_beta, gn1_alpha),
           (gn2_gamma, gn2_beta, gn2_alpha),
           (gn3_gamma, gn3_beta, gn3_alpha))
    phis = ((r1_phi_w, r1_phi_b), (r2_phi_w, r2_phi_b), (r3_phi_w, r3_phi_b))
    ws = (W1, W2, W3)

    h = node_feats
    wm0 = None
    wms, pps = [], []
    for l in range(3):
        hm, wm = _tc_a(h, dvo, nwc, ars[0], ws[l])
        if l == 0:
            wm0 = wm
        part = _prop_call(hm, srcr, dstr, ewr, spr[l])
        agg, st = _tc_b1(part, dvi)
        g2, b2, a2 = (g.reshape(1, _F) for g in gns[l])
        pw, pb = phis[l][0], phis[l][1].reshape(1, _R)
        h, pp, wmk = _tc_b2(agg, st, g2, b2, a2, nwc, ars[l + 1], pw, pb)
        wms.append(wmk)
        pps.append(pp)

    return _tc_final(wm0, wms[0], wms[1], wms[2], pps[0], pps[1], pps[2],
                     (r1_rho_w, r2_rho_w, r3_rho_w),
                     (r1_rho_b.reshape(1, _R), r2_rho_b.reshape(1, _R),
                      r3_rho_b.reshape(1, _R)))


# speedup vs baseline: 9.5116x; 1.8875x over previous
import jax
import jax.numpy as jnp
from jax import lax
from jax.experimental import pallas as pl
from jax.experimental.pallas import tpu as pltpu
from jax.experimental.pallas import tpu_sc as plsc

_N = 10000
_E = 320000
_F = 128
_R = 64
_NPAD = 10240
_NC = 2
_NS = 16
_NW = _NC * _NS
_ER = _E // 128
_RPW = _ER // _NW
_REM = _ER - _RPW * _NW
_WR = 2
_NWIN = _RPW // _WR
_BN = 2000


def _leaky(x):
    return jnp.where(x >= 0, x, 0.01 * x)


def _deg_body(src1, dst1, out_hbm, ho, hi, sbuf, dbuf):
    c = lax.axis_index("c")
    s = lax.axis_index("s")
    wid = c * _NS + s

    def _z(i, _):
        ho[pl.ds(i * 16, 16)] = jnp.zeros((16,), jnp.float32)
        hi[pl.ds(i * 16, 16)] = jnp.zeros((16,), jnp.float32)
        return 0

    lax.fori_loop(0, _NPAD // 16, _z, 0)

    base = (wid * _RPW + jnp.minimum(wid, _REM)) * 128
    ne = _RPW * 128
    pltpu.sync_copy(src1.at[pl.ds(base, ne)], sbuf.at[pl.ds(0, ne)])
    pltpu.sync_copy(dst1.at[pl.ds(base, ne)], dbuf.at[pl.ds(0, ne)])

    @pl.when(wid < _REM)
    def _():
        pltpu.sync_copy(src1.at[pl.ds(base + ne, 128)],
                        sbuf.at[pl.ds(ne, 128)])
        pltpu.sync_copy(dst1.at[pl.ds(base + ne, 128)],
                        dbuf.at[pl.ds(ne, 128)])

    ones = jnp.full((16,), 1.0, jnp.float32)

    def _sc(i, _):
        iv = sbuf[pl.ds(i * 16, 16)]
        plsc.addupdate_scatter(ho, [iv], ones)
        jv = dbuf[pl.ds(i * 16, 16)]
        plsc.addupdate_scatter(hi, [jv], ones)
        return 0

    lax.fori_loop(0, ne // 16, _sc, 0)

    @pl.when(wid < _REM)
    def _():
        lax.fori_loop(ne // 16, ne // 16 + 8, _sc, 0)

    pltpu.sync_copy(ho, out_hbm.at[pl.ds(wid * _NPAD, _NPAD)])
    pltpu.sync_copy(hi, out_hbm.at[pl.ds((_NW + wid) * _NPAD, _NPAD)])


def _prop_body(h_hbm, src1, dst1, ew1, sp1, out_hbm,
               acc, sbufs, dbufs, ewbs, spbs, rows, semi, semg, semsc):
    c = lax.axis_index("c")
    s = lax.axis_index("s")

    def _z(i, _):
        r = i // 8
        k = i % 8
        rows[0][r, pl.ds(k * 16, 16)] = jnp.zeros((16,), jnp.float32)
        return 0

    lax.fori_loop(0, 128 * 8, _z, 0)

    def _za(bk, _):
        pltpu.sync_copy(rows[0], acc.at[pl.ds(s * 624 + bk * 128, 128)])
        return 0

    lax.fori_loop(0, 4, _za, 0)
    pltpu.sync_copy(rows[0].at[pl.ds(0, 112)],
                    acc.at[pl.ds(s * 624 + 512, 112)])

    @pl.when(s == _NS - 1)
    def _():
        pltpu.sync_copy(rows[0].at[pl.ds(0, 16)], acc.at[pl.ds(9984, 16)])

    plsc.subcore_barrier()

    base_g = c * 1250 + s * 78 + jnp.minimum(s, 2)

    def _e0(w):
        return (base_g + w) * 128

    def _prefetch(w, m):
        e0 = _e0(w)
        pltpu.async_copy(src1.at[pl.ds(e0, 128)], sbufs[m], semi[m])
        pltpu.async_copy(dst1.at[pl.ds(e0, 128)], dbufs[m], semi[m])
        pltpu.async_copy(ew1.at[pl.ds(e0, 128)], ewbs[m], semi[m])
        pltpu.async_copy(sp1.at[pl.ds(e0, 128)], spbs[m], semi[m])

    def _wait_prefetch(w, m):
        e0 = _e0(w)
        pltpu.make_async_copy(src1.at[pl.ds(e0, 128)], sbufs[m], semi[m]).wait()
        pltpu.make_async_copy(dst1.at[pl.ds(e0, 128)], dbufs[m], semi[m]).wait()
        pltpu.make_async_copy(ew1.at[pl.ds(e0, 128)], ewbs[m], semi[m]).wait()
        pltpu.make_async_copy(sp1.at[pl.ds(e0, 128)], spbs[m], semi[m]).wait()

    def _gather(m, b):
        pltpu.async_copy(h_hbm.at[sbufs[m]], rows[b], semg[b])

    def _wait_gather(m, b):
        pltpu.make_async_copy(h_hbm.at[sbufs[m]], rows[b], semg[b]).wait()

    def _scale(m, b):
        def _grp(g, _):
            wv = ewbs[m][pl.ds(g * 16, 16)] * spbs[m][pl.ds(g * 16, 16)]
            for q in range(16):
                w_ = wv[q]
                r = g * 16 + q
                for f in range(8):
                    sl = pl.ds(f * 16, 16)
                    rows[b][r, sl] = rows[b][r, sl] * w_
            return 0

        lax.fori_loop(0, 8, _grp, 0)

    def _scatter(m, b):
        pltpu.async_copy(rows[b], acc.at[dbufs[m]], semsc[b], add=True)

    def _wait_scatter(m, b):
        pltpu.make_async_copy(rows[b], acc.at[dbufs[m]], semsc[b]).wait()

    def _win(w, t, guard_first, do_next, do_next2):
        b = t % 2
        m = t % 3
        _wait_gather(m, b)
        if guard_first:
            @pl.when(w >= 1)
            def _():
                _wait_scatter((t - 1) % 3, (t - 1) % 2)
        else:
            _wait_scatter((t - 1) % 3, (t - 1) % 2)
        if do_next:
            _wait_prefetch(w + 1, (t + 1) % 3)
            _gather((t + 1) % 3, (t + 1) % 2)
        if do_next2:
            _prefetch(w + 2, (t + 2) % 3)
        _scale(m, b)
        _scatter(m, b)

    _prefetch(0, 0)
    _prefetch(1, 1)
    _wait_prefetch(0, 0)
    _gather(0, 0)

    def _sb(k, _):
        w0 = k * 6
        for t in range(6):
            _win(w0 + t, t, guard_first=(t == 0), do_next=True, do_next2=True)
        return 0

    lax.fori_loop(0, 12, _sb, 0)

    for t in range(6):
        w = 72 + t
        _win(w, t, guard_first=False,
             do_next=(w + 1 < 78), do_next2=(w + 2 < 78))
    _wait_scatter(5 % 3, 5 % 2)

    @pl.when(s < 2)
    def _():
        e0 = _e0(78)
        pltpu.sync_copy(src1.at[pl.ds(e0, 128)], sbufs[0])
        pltpu.sync_copy(dst1.at[pl.ds(e0, 128)], dbufs[0])
        pltpu.sync_copy(ew1.at[pl.ds(e0, 128)], ewbs[0])
        pltpu.sync_copy(sp1.at[pl.ds(e0, 128)], spbs[0])
        _gather(0, 0)
        _wait_gather(0, 0)
        _scale(0, 0)
        _scatter(0, 0)
        _wait_scatter(0, 0)

    plsc.subcore_barrier()
    pltpu.sync_copy(acc.at[pl.ds(s * 624, 624)],
                    out_hbm.at[c, pl.ds(s * 624, 624)])

    @pl.when(s == _NS - 1)
    def _():
        pltpu.sync_copy(acc.at[pl.ds(9984, 16)],
                        out_hbm.at[c, pl.ds(9984, 16)])


_sc_kernels = {}


def _get_sc_kernels():
    if "deg" not in _sc_kernels:
        mesh = plsc.VectorSubcoreMesh(
            core_axis_name="c", subcore_axis_name="s",
            num_cores=_NC, num_subcores=_NS)
        _sc_kernels["deg"] = pl.kernel(
            _deg_body,
            out_type=jax.ShapeDtypeStruct((2 * _NW * _NPAD,), jnp.float32),
            mesh=mesh,
            compiler_params=pltpu.CompilerParams(needs_layout_passes=False),
            scratch_types=[
                pltpu.VMEM((_NPAD,), jnp.float32),
                pltpu.VMEM((_NPAD,), jnp.float32),
                pltpu.VMEM(((_RPW + 1) * 128,), jnp.int32),
                pltpu.VMEM(((_RPW + 1) * 128,), jnp.int32),
            ],
        )
        _sc_kernels["prop"] = pl.kernel(
            _prop_body,
            out_type=jax.ShapeDtypeStruct((2, _N, _F), jnp.float32),
            mesh=mesh,
            scratch_types=[
                pltpu.VMEM_SHARED((_N, _F), jnp.float32),
                [pltpu.VMEM((128,), jnp.int32) for _ in range(3)],
                [pltpu.VMEM((128,), jnp.int32) for _ in range(3)],
                [pltpu.VMEM((128,), jnp.float32) for _ in range(3)],
                [pltpu.VMEM((128,), jnp.float32) for _ in range(3)],
                [pltpu.VMEM((128, 128), jnp.float32) for _ in range(2)],
                [pltpu.SemaphoreType.DMA for _ in range(3)],
                [pltpu.SemaphoreType.DMA for _ in range(2)],
                [pltpu.SemaphoreType.DMA for _ in range(2)],
            ],
        )
    return _sc_kernels


def _deg_call(srcr, dstr):
    return _get_sc_kernels()["deg"](srcr, dstr)


def _prop_call(hm, srcr, dstr, ewr, spr):
    return _get_sc_kernels()["prop"](hm, srcr, dstr, ewr, spr)


def _prep_body(h_ref, o_ref):
    x = h_ref[...]
    dego = jnp.sum(x[0:_NW], axis=0, keepdims=True)
    degi = jnp.sum(x[_NW:2 * _NW], axis=0, keepdims=True)
    deg = jnp.concatenate([dego, degi], axis=0)
    o_ref[...] = jax.lax.rsqrt(jnp.maximum(deg, 1.0))


def _a_body(x_ref, d_ref, nw_ref, ar_ref, w_ref, h_ref, ws_ref):
    i = pl.program_id(0)
    x = x_ref[...]
    h_ref[...] = jnp.dot(x * d_ref[...], w_ref[...],
                         preferred_element_type=jnp.float32)
    wm = jnp.sum(x * (nw_ref[...] * ar_ref[...]), axis=0, keepdims=True)

    @pl.when(i == 0)
    def _():
        ws_ref[...] = wm

    @pl.when(i != 0)
    def _():
        ws_ref[...] += wm


def _b1_body(p_ref, d_ref, agg_ref, st_ref):
    i = pl.program_id(0)
    p = p_ref[...]
    sgg = (p[0] + p[1]) * d_ref[...]
    agg_ref[...] = sgg
    st = jnp.concatenate(
        [jnp.sum(sgg, axis=0, keepdims=True),
         jnp.sum(sgg * sgg, axis=0, keepdims=True)], axis=0)

    @pl.when(i == 0)
    def _():
        st_ref[...] = st

    @pl.when(i != 0)
    def _():
        st_ref[...] += st


def _b2_body(agg_ref, st_ref, g_ref, b_ref, al_ref, nw_ref, ar_ref,
             pw_ref, pb_ref, h_ref, pp_ref, ws_ref):
    i = pl.program_id(0)
    st = st_ref[...]
    m = st[0:1] / _N
    msq = st[1:2] / _N
    al = al_ref[...]
    var = msq - (2.0 * al - al * al) * (m * m)
    stdv = jnp.sqrt(var + 1e-5)
    a = agg_ref[...]
    xn = g_ref[...] * (a - al * m) / stdv + b_ref[...]
    hl = _leaky(xn)
    h_ref[...] = hl
    phi = _leaky(jnp.dot(hl, pw_ref[...],
                         preferred_element_type=jnp.float32) + pb_ref[...])
    pp = jnp.sum(phi, axis=0, keepdims=True)
    wm = jnp.sum(hl * (nw_ref[...] * ar_ref[...]), axis=0, keepdims=True)

    @pl.when(i == 0)
    def _():
        pp_ref[...] = pp
        ws_ref[...] = wm

    @pl.when(i != 0)
    def _():
        pp_ref[...] += pp
        ws_ref[...] += wm


def _f_body(wm0, wm1, wm2, wm3, pp1, pp2, pp3,
            rw1, rw2, rw3, rb1, rb2, rb3, out_ref):
    rs = []
    for pp, rw, rb in ((pp1, rw1, rb1), (pp2, rw2, rb2), (pp3, rw3, rb3)):
        rs.append(_leaky(jnp.dot(pp[...] / _N, rw[...],
                                 preferred_element_type=jnp.float32) + rb[...]))
    out_ref[...] = _leaky(jnp.concatenate(
        [wm0[...] / _N, rs[0], wm1[...] / _N, rs[1],
         wm2[...] / _N, rs[2], wm3[...] / _N], axis=1))


def _tc_prep(hists):
    return pl.pallas_call(
        _prep_body,
        out_shape=jax.ShapeDtypeStruct((2, _NPAD), jnp.float32),
    )(hists)


def _tc_a(x, dvo, nwc, arc, w):
    g = _N // _BN
    return pl.pallas_call(
        _a_body,
        grid=(g,),
        in_specs=[
            pl.BlockSpec((_BN, _F), lambda i: (i, 0)),
            pl.BlockSpec((_BN, 1), lambda i: (i, 0)),
            pl.BlockSpec((_BN, 1), lambda i: (i, 0)),
            pl.BlockSpec((_BN, 1), lambda i: (i, 0)),
            pl.BlockSpec((_F, _F), lambda i: (0, 0)),
        ],
        out_specs=[
            pl.BlockSpec((_BN, _F), lambda i: (i, 0)),
            pl.BlockSpec((1, _F), lambda i: (0, 0)),
        ],
        out_shape=[
            jax.ShapeDtypeStruct((_N, _F), jnp.float32),
            jax.ShapeDtypeStruct((1, _F), jnp.float32),
        ],
    )(x, dvo, nwc, arc, w)


def _tc_b1(p, dvi):
    g = _N // _BN
    return pl.pallas_call(
        _b1_body,
        grid=(g,),
        in_specs=[
            pl.BlockSpec((2, _BN, _F), lambda i: (0, i, 0)),
            pl.BlockSpec((_BN, 1), lambda i: (i, 0)),
        ],
        out_specs=[
            pl.BlockSpec((_BN, _F), lambda i: (i, 0)),
            pl.BlockSpec((2, _F), lambda i: (0, 0)),
        ],
        out_shape=[
            jax.ShapeDtypeStruct((_N, _F), jnp.float32),
            jax.ShapeDtypeStruct((2, _F), jnp.float32),
        ],
    )(p, dvi)


def _tc_b2(agg, st, g2, b2, a2, nwc, arc, pw, pb):
    g = _N // _BN
    return pl.pallas_call(
        _b2_body,
        grid=(g,),
        in_specs=[
            pl.BlockSpec((_BN, _F), lambda i: (i, 0)),
            pl.BlockSpec((2, _F), lambda i: (0, 0)),
            pl.BlockSpec((1, _F), lambda i: (0, 0)),
            pl.BlockSpec((1, _F), lambda i: (0, 0)),
            pl.BlockSpec((1, _F), lambda i: (0, 0)),
            pl.BlockSpec((_BN, 1), lambda i: (i, 0)),
            pl.BlockSpec((_BN, 1), lambda i: (i, 0)),
            pl.BlockSpec((_F, _R), lambda i: (0, 0)),
            pl.BlockSpec((1, _R), lambda i: (0, 0)),
        ],
        out_specs=[
            pl.BlockSpec((_BN, _F), lambda i: (i, 0)),
            pl.BlockSpec((1, _R), lambda i: (0, 0)),
            pl.BlockSpec((1, _F), lambda i: (0, 0)),
        ],
        out_shape=[
            jax.ShapeDtypeStruct((_N, _F), jnp.float32),
            jax.ShapeDtypeStruct((1, _R), jnp.float32),
            jax.ShapeDtypeStruct((1, _F), jnp.float32),
        ],
    )(agg, st, g2, b2, a2, nwc, arc, pw, pb)


def _tc_final(wm0, wm1, wm2, wm3, pp1, pp2, pp3, rws, rbs):
    return pl.pallas_call(
        _f_body,
        out_shape=jax.ShapeDtypeStruct((1, 704), jnp.float32),
    )(wm0, wm1, wm2, wm3, pp1, pp2, pp3,
      rws[0], rws[1], rws[2], rbs[0], rbs[1], rbs[2])


def kernel(node_feats, edge_weights, node_weights, suppl1, suppl2, suppl3,
           AR1, AR2, AR3, AR4, W1, W2, W3,
           gn1_gamma, gn1_beta, gn1_alpha,
           gn2_gamma, gn2_beta, gn2_alpha,
           gn3_gamma, gn3_beta, gn3_alpha,
           r1_phi_w, r1_phi_b, r1_rho_w, r1_rho_b,
           r2_phi_w, r2_phi_b, r2_rho_w, r2_rho_b,
           r3_phi_w, r3_phi_b, r3_rho_w, r3_rho_b, edge_index):
    srcr = edge_index[0]
    dstr = edge_index[1]
    ewr = edge_weights
    spr = (suppl1, suppl2, suppl3)

    hists = _deg_call(srcr, dstr).reshape(2 * _NW, _NPAD)
    dinv2 = _tc_prep(hists)
    dvo = dinv2[0, :_N].reshape(_N, 1)
    dvi = dinv2[1, :_N].reshape(_N, 1)
    nwc = node_weights.reshape(_N, 1)
    ars = (AR1.reshape(_N, 1), AR2.reshape(_N, 1),
           AR3.reshape(_N, 1), AR4.reshape(_N, 1))
    gns = ((gn1_gamma, gn1_beta, gn1_alpha),
           (gn2_gamma, gn2_beta, gn2_alpha),
           (gn3_gamma, gn3_beta, gn3_alpha))
    phis = ((r1_phi_w, r1_phi_b), (r2_phi_w, r2_phi_b), (r3_phi_w, r3_phi_b))
    ws = (W1, W2, W3)

    h = node_feats
    wm0 = None
    wms, pps = [], []
    for l in range(3):
        hm, wm = _tc_a(h, dvo, nwc, ars[0], ws[l])
        if l == 0:
            wm0 = wm
        part = _prop_call(hm, srcr, dstr, ewr, spr[l])
        agg, st = _tc_b1(part, dvi)
        g2, b2, a2 = (g.reshape(1, _F) for g in gns[l])
        pw, pb = phis[l][0], phis[l][1].reshape(1, _R)
        h, pp, wmk = _tc_b2(agg, st, g2, b2, a2, nwc, ars[l + 1], pw, pb)
        wms.append(wmk)
        pps.append(pp)

    return _tc_final(wm0, wms[0], wms[1], wms[2], pps[0], pps[1], pps[2],
                     (r1_rho_w, r2_rho_w, r3_rho_w),
                     (r1_rho_b.reshape(1, _R), r2_rho_b.reshape(1, _R),
                      r3_rho_b.reshape(1, _R)))


# speedup vs baseline: 9.6011x; 1.0094x over previous
import jax
import jax.numpy as jnp
from jax import lax
from jax.experimental import pallas as pl
from jax.experimental.pallas import tpu as pltpu
from jax.experimental.pallas import tpu_sc as plsc

_N = 10000
_E = 320000
_F = 128
_R = 64
_NPAD = 10240
_NC = 2
_NS = 16
_NW = _NC * _NS
_ER = _E // 128
_RPW = _ER // _NW
_REM = _ER - _RPW * _NW
_WR = 2
_NWIN = _RPW // _WR
_BN = 2000


def _leaky(x):
    return jnp.where(x >= 0, x, 0.01 * x)


def _deg_body(src1, dst1, out_hbm, ho, hi, sbuf, dbuf):
    c = lax.axis_index("c")
    s = lax.axis_index("s")
    wid = c * _NS + s

    def _z(i, _):
        ho[pl.ds(i * 16, 16)] = jnp.zeros((16,), jnp.float32)
        hi[pl.ds(i * 16, 16)] = jnp.zeros((16,), jnp.float32)
        return 0

    lax.fori_loop(0, _NPAD // 16, _z, 0)

    base = (wid * _RPW + jnp.minimum(wid, _REM)) * 128
    ne = _RPW * 128
    pltpu.sync_copy(src1.at[pl.ds(base, ne)], sbuf.at[pl.ds(0, ne)])
    pltpu.sync_copy(dst1.at[pl.ds(base, ne)], dbuf.at[pl.ds(0, ne)])

    @pl.when(wid < _REM)
    def _():
        pltpu.sync_copy(src1.at[pl.ds(base + ne, 128)],
                        sbuf.at[pl.ds(ne, 128)])
        pltpu.sync_copy(dst1.at[pl.ds(base + ne, 128)],
                        dbuf.at[pl.ds(ne, 128)])

    ones = jnp.full((16,), 1.0, jnp.float32)

    def _sc(i, _):
        iv = sbuf[pl.ds(i * 16, 16)]
        plsc.addupdate_scatter(ho, [iv], ones)
        jv = dbuf[pl.ds(i * 16, 16)]
        plsc.addupdate_scatter(hi, [jv], ones)
        return 0

    lax.fori_loop(0, ne // 16, _sc, 0)

    @pl.when(wid < _REM)
    def _():
        lax.fori_loop(ne // 16, ne // 16 + 8, _sc, 0)

    pltpu.sync_copy(ho, out_hbm.at[pl.ds(wid * _NPAD, _NPAD)])
    pltpu.sync_copy(hi, out_hbm.at[pl.ds((_NW + wid) * _NPAD, _NPAD)])


def _prop_body(h_hbm, src1, dst1, ew1, out_hbm,
               acc, sbufs, dbufs, ewbs, rows, semi, semg, semsc):
    c = lax.axis_index("c")
    s = lax.axis_index("s")

    def _z(i, _):
        r = i // 8
        k = i % 8
        rows[0][r, pl.ds(k * 16, 16)] = jnp.zeros((16,), jnp.float32)
        return 0

    lax.fori_loop(0, 128 * 8, _z, 0)

    def _za(bk, _):
        pltpu.sync_copy(rows[0], acc.at[pl.ds(s * 624 + bk * 128, 128)])
        return 0

    lax.fori_loop(0, 4, _za, 0)
    pltpu.sync_copy(rows[0].at[pl.ds(0, 112)],
                    acc.at[pl.ds(s * 624 + 512, 112)])

    @pl.when(s == _NS - 1)
    def _():
        pltpu.sync_copy(rows[0].at[pl.ds(0, 16)], acc.at[pl.ds(9984, 16)])

    plsc.subcore_barrier()

    base_g = c * 1250 + s * 78 + jnp.minimum(s, 2)

    def _e0(w):
        return (base_g + w) * 128

    def _prefetch(w, m):
        e0 = _e0(w)
        pltpu.async_copy(src1.at[pl.ds(e0, 128)], sbufs[m], semi[m])
        pltpu.async_copy(dst1.at[pl.ds(e0, 128)], dbufs[m], semi[m])
        pltpu.async_copy(ew1.at[pl.ds(e0, 128)], ewbs[m], semi[m])

    def _wait_prefetch(w, m):
        e0 = _e0(w)
        pltpu.make_async_copy(src1.at[pl.ds(e0, 128)], sbufs[m], semi[m]).wait()
        pltpu.make_async_copy(dst1.at[pl.ds(e0, 128)], dbufs[m], semi[m]).wait()
        pltpu.make_async_copy(ew1.at[pl.ds(e0, 128)], ewbs[m], semi[m]).wait()

    def _gather(m, b):
        pltpu.async_copy(h_hbm.at[sbufs[m]], rows[b], semg[b])

    def _wait_gather(m, b):
        pltpu.make_async_copy(h_hbm.at[sbufs[m]], rows[b], semg[b]).wait()

    def _scale(m, b):
        def _grp(g, _):
            wv = ewbs[m][pl.ds(g * 16, 16)]
            for q in range(16):
                w_ = wv[q]
                r = g * 16 + q
                for f in range(8):
                    sl = pl.ds(f * 16, 16)
                    rows[b][r, sl] = rows[b][r, sl] * w_
            return 0

        lax.fori_loop(0, 8, _grp, 0)

    def _scatter(m, b):
        pltpu.async_copy(rows[b], acc.at[dbufs[m]], semsc[b], add=True)

    def _wait_scatter(m, b):
        pltpu.make_async_copy(rows[b], acc.at[dbufs[m]], semsc[b]).wait()

    def _win(w, t, guard_first, la1, la2):
        b = t % 3
        m = t % 4
        _wait_gather(m, b)
        if guard_first:
            @pl.when(w >= 1)
            def _():
                _wait_scatter((t - 1) % 4, (t - 1) % 3)
        else:
            _wait_scatter((t - 1) % 4, (t - 1) % 3)
        if la2:
            _wait_prefetch(w + 2, (t + 2) % 4)
            _gather((t + 2) % 4, (t + 2) % 3)
        if la1:
            _prefetch(w + 3, (t + 3) % 4)
        _scale(m, b)
        _scatter(m, b)

    _prefetch(0, 0)
    _prefetch(1, 1)
    _prefetch(2, 2)
    _wait_prefetch(0, 0)
    _gather(0, 0)
    _wait_prefetch(1, 1)
    _gather(1, 1)

    def _sb(k, _):
        w0 = k * 12
        for t in range(12):
            _win(w0 + t, t, guard_first=(t <= 1), la1=True, la2=True)
        return 0

    lax.fori_loop(0, 6, _sb, 0)

    for t in range(6):
        w = 72 + t
        _win(w, t, guard_first=False,
             la1=(w + 3 < 78), la2=(w + 2 < 78))
    _wait_scatter(77 % 4, 77 % 3)

    @pl.when(s < 2)
    def _():
        e0 = _e0(78)
        pltpu.sync_copy(src1.at[pl.ds(e0, 128)], sbufs[0])
        pltpu.sync_copy(dst1.at[pl.ds(e0, 128)], dbufs[0])
        pltpu.sync_copy(ew1.at[pl.ds(e0, 128)], ewbs[0])
        _gather(0, 0)
        _wait_gather(0, 0)
        _scale(0, 0)
        _scatter(0, 0)
        _wait_scatter(0, 0)

    plsc.subcore_barrier()
    pltpu.sync_copy(acc.at[pl.ds(s * 624, 624)],
                    out_hbm.at[c, pl.ds(s * 624, 624)])

    @pl.when(s == _NS - 1)
    def _():
        pltpu.sync_copy(acc.at[pl.ds(9984, 16)],
                        out_hbm.at[c, pl.ds(9984, 16)])


_sc_kernels = {}


def _get_sc_kernels():
    if "deg" not in _sc_kernels:
        mesh = plsc.VectorSubcoreMesh(
            core_axis_name="c", subcore_axis_name="s",
            num_cores=_NC, num_subcores=_NS)
        _sc_kernels["deg"] = pl.kernel(
            _deg_body,
            out_type=jax.ShapeDtypeStruct((2 * _NW * _NPAD,), jnp.float32),
            mesh=mesh,
            compiler_params=pltpu.CompilerParams(needs_layout_passes=False),
            scratch_types=[
                pltpu.VMEM((_NPAD,), jnp.float32),
                pltpu.VMEM((_NPAD,), jnp.float32),
                pltpu.VMEM(((_RPW + 1) * 128,), jnp.int32),
                pltpu.VMEM(((_RPW + 1) * 128,), jnp.int32),
            ],
        )
        _sc_kernels["prop"] = pl.kernel(
            _prop_body,
            out_type=jax.ShapeDtypeStruct((2, _N, _F), jnp.float32),
            mesh=mesh,
            scratch_types=[
                pltpu.VMEM_SHARED((_N, _F), jnp.float32),
                [pltpu.VMEM((128,), jnp.int32) for _ in range(4)],
                [pltpu.VMEM((128,), jnp.int32) for _ in range(4)],
                [pltpu.VMEM((128,), jnp.float32) for _ in range(4)],
                [pltpu.VMEM((128, 128), jnp.float32) for _ in range(3)],
                [pltpu.SemaphoreType.DMA for _ in range(4)],
                [pltpu.SemaphoreType.DMA for _ in range(3)],
                [pltpu.SemaphoreType.DMA for _ in range(3)],
            ],
        )
    return _sc_kernels


def _deg_call(srcr, dstr):
    return _get_sc_kernels()["deg"](srcr, dstr)


def _prop_call(hm, srcr, dstr, ewer):
    return _get_sc_kernels()["prop"](hm, srcr, dstr, ewer)


def _prep_body(h_ref, o_ref):
    x = h_ref[...]
    dego = jnp.sum(x[0:_NW], axis=0, keepdims=True)
    degi = jnp.sum(x[_NW:2 * _NW], axis=0, keepdims=True)
    deg = jnp.concatenate([dego, degi], axis=0)
    o_ref[...] = jax.lax.rsqrt(jnp.maximum(deg, 1.0))


def _a_body(x_ref, d_ref, nw_ref, ar_ref, w_ref, ew_ref, sp_ref,
            h_ref, ws_ref, ewe_ref):
    i = pl.program_id(0)
    x = x_ref[...]
    h_ref[...] = jnp.dot(x * d_ref[...], w_ref[...],
                         preferred_element_type=jnp.float32)
    ewe_ref[...] = ew_ref[...] * sp_ref[...]
    wm = jnp.sum(x * (nw_ref[...] * ar_ref[...]), axis=0, keepdims=True)

    @pl.when(i == 0)
    def _():
        ws_ref[...] = wm

    @pl.when(i != 0)
    def _():
        ws_ref[...] += wm


def _b1_body(p_ref, d_ref, agg_ref, st_ref):
    i = pl.program_id(0)
    p = p_ref[...]
    sgg = (p[0] + p[1]) * d_ref[...]
    agg_ref[...] = sgg
    st = jnp.concatenate(
        [jnp.sum(sgg, axis=0, keepdims=True),
         jnp.sum(sgg * sgg, axis=0, keepdims=True)], axis=0)

    @pl.when(i == 0)
    def _():
        st_ref[...] = st

    @pl.when(i != 0)
    def _():
        st_ref[...] += st


def _b2_body(agg_ref, st_ref, g_ref, b_ref, al_ref, nw_ref, ar_ref,
             pw_ref, pb_ref, h_ref, pp_ref, ws_ref):
    i = pl.program_id(0)
    st = st_ref[...]
    m = st[0:1] / _N
    msq = st[1:2] / _N
    al = al_ref[...]
    var = msq - (2.0 * al - al * al) * (m * m)
    stdv = jnp.sqrt(var + 1e-5)
    a = agg_ref[...]
    xn = g_ref[...] * (a - al * m) / stdv + b_ref[...]
    hl = _leaky(xn)
    h_ref[...] = hl
    phi = _leaky(jnp.dot(hl, pw_ref[...],
                         preferred_element_type=jnp.float32) + pb_ref[...])
    pp = jnp.sum(phi, axis=0, keepdims=True)
    wm = jnp.sum(hl * (nw_ref[...] * ar_ref[...]), axis=0, keepdims=True)

    @pl.when(i == 0)
    def _():
        pp_ref[...] = pp
        ws_ref[...] = wm

    @pl.when(i != 0)
    def _():
        pp_ref[...] += pp
        ws_ref[...] += wm


def _f_body(wm0, wm1, wm2, wm3, pp1, pp2, pp3,
            rw1, rw2, rw3, rb1, rb2, rb3, out_ref):
    rs = []
    for pp, rw, rb in ((pp1, rw1, rb1), (pp2, rw2, rb2), (pp3, rw3, rb3)):
        rs.append(_leaky(jnp.dot(pp[...] / _N, rw[...],
                                 preferred_element_type=jnp.float32) + rb[...]))
    out_ref[...] = _leaky(jnp.concatenate(
        [wm0[...] / _N, rs[0], wm1[...] / _N, rs[1],
         wm2[...] / _N, rs[2], wm3[...] / _N], axis=1))


def _tc_prep(hists):
    return pl.pallas_call(
        _prep_body,
        out_shape=jax.ShapeDtypeStruct((2, _NPAD), jnp.float32),
    )(hists)


def _tc_a(x, dvo, nwc, arc, w, ew2, sp2):
    g = _N // _BN
    eb = _E // g
    return pl.pallas_call(
        _a_body,
        grid=(g,),
        in_specs=[
            pl.BlockSpec((_BN, _F), lambda i: (i, 0)),
            pl.BlockSpec((_BN, 1), lambda i: (i, 0)),
            pl.BlockSpec((_BN, 1), lambda i: (i, 0)),
            pl.BlockSpec((_BN, 1), lambda i: (i, 0)),
            pl.BlockSpec((_F, _F), lambda i: (0, 0)),
            pl.BlockSpec((1, 1, eb), lambda i: (i, 0, 0)),
            pl.BlockSpec((1, 1, eb), lambda i: (i, 0, 0)),
        ],
        out_specs=[
            pl.BlockSpec((_BN, _F), lambda i: (i, 0)),
            pl.BlockSpec((1, _F), lambda i: (0, 0)),
            pl.BlockSpec((1, 1, eb), lambda i: (i, 0, 0)),
        ],
        out_shape=[
            jax.ShapeDtypeStruct((_N, _F), jnp.float32),
            jax.ShapeDtypeStruct((1, _F), jnp.float32),
            jax.ShapeDtypeStruct((g, 1, eb), jnp.float32),
        ],
    )(x, dvo, nwc, arc, w, ew2, sp2)


def _tc_b1(p, dvi):
    g = _N // _BN
    return pl.pallas_call(
        _b1_body,
        grid=(g,),
        in_specs=[
            pl.BlockSpec((2, _BN, _F), lambda i: (0, i, 0)),
            pl.BlockSpec((_BN, 1), lambda i: (i, 0)),
        ],
        out_specs=[
            pl.BlockSpec((_BN, _F), lambda i: (i, 0)),
            pl.BlockSpec((2, _F), lambda i: (0, 0)),
        ],
        out_shape=[
            jax.ShapeDtypeStruct((_N, _F), jnp.float32),
            jax.ShapeDtypeStruct((2, _F), jnp.float32),
        ],
    )(p, dvi)


def _tc_b2(agg, st, g2, b2, a2, nwc, arc, pw, pb):
    g = _N // _BN
    return pl.pallas_call(
        _b2_body,
        grid=(g,),
        in_specs=[
            pl.BlockSpec((_BN, _F), lambda i: (i, 0)),
            pl.BlockSpec((2, _F), lambda i: (0, 0)),
            pl.BlockSpec((1, _F), lambda i: (0, 0)),
            pl.BlockSpec((1, _F), lambda i: (0, 0)),
            pl.BlockSpec((1, _F), lambda i: (0, 0)),
            pl.BlockSpec((_BN, 1), lambda i: (i, 0)),
            pl.BlockSpec((_BN, 1), lambda i: (i, 0)),
            pl.BlockSpec((_F, _R), lambda i: (0, 0)),
            pl.BlockSpec((1, _R), lambda i: (0, 0)),
        ],
        out_specs=[
            pl.BlockSpec((_BN, _F), lambda i: (i, 0)),
            pl.BlockSpec((1, _R), lambda i: (0, 0)),
            pl.BlockSpec((1, _F), lambda i: (0, 0)),
        ],
        out_shape=[
            jax.ShapeDtypeStruct((_N, _F), jnp.float32),
            jax.ShapeDtypeStruct((1, _R), jnp.float32),
            jax.ShapeDtypeStruct((1, _F), jnp.float32),
        ],
    )(agg, st, g2, b2, a2, nwc, arc, pw, pb)


def _tc_final(wm0, wm1, wm2, wm3, pp1, pp2, pp3, rws, rbs):
    return pl.pallas_call(
        _f_body,
        out_shape=jax.ShapeDtypeStruct((1, 704), jnp.float32),
    )(wm0, wm1, wm2, wm3, pp1, pp2, pp3,
      rws[0], rws[1], rws[2], rbs[0], rbs[1], rbs[2])


def kernel(node_feats, edge_weights, node_weights, suppl1, suppl2, suppl3,
           AR1, AR2, AR3, AR4, W1, W2, W3,
           gn1_gamma, gn1_beta, gn1_alpha,
           gn2_gamma, gn2_beta, gn2_alpha,
           gn3_gamma, gn3_beta, gn3_alpha,
           r1_phi_w, r1_phi_b, r1_rho_w, r1_rho_b,
           r2_phi_w, r2_phi_b, r2_rho_w, r2_rho_b,
           r3_phi_w, r3_phi_b, r3_rho_w, r3_rho_b, edge_index):
    srcr = edge_index[0]
    dstr = edge_index[1]
    g = _N // _BN
    eb = _E // g
    ew2 = edge_weights.reshape(g, 1, eb)
    sp2 = (suppl1.reshape(g, 1, eb), suppl2.reshape(g, 1, eb),
           suppl3.reshape(g, 1, eb))

    hists = _deg_call(srcr, dstr).reshape(2 * _NW, _NPAD)
    dinv2 = _tc_prep(hists)
    dvo = dinv2[0, :_N].reshape(_N, 1)
    dvi = dinv2[1, :_N].reshape(_N, 1)
    nwc = node_weights.reshape(_N, 1)
    ars = (AR1.reshape(_N, 1), AR2.reshape(_N, 1),
           AR3.reshape(_N, 1), AR4.reshape(_N, 1))
    gns = ((gn1_gamma, gn1_beta, gn1_alpha),
           (gn2_gamma, gn2_beta, gn2_alpha),
           (gn3_gamma, gn3_beta, gn3_alpha))
    phis = ((r1_phi_w, r1_phi_b), (r2_phi_w, r2_phi_b), (r3_phi_w, r3_phi_b))
    ws = (W1, W2, W3)

    h = node_feats
    wm0 = None
    wms, pps = [], []
    for l in range(3):
        hm, wm, ewe = _tc_a(h, dvo, nwc, ars[0], ws[l], ew2, sp2[l])
        if l == 0:
            wm0 = wm
        part = _prop_call(hm, srcr, dstr, ewe.reshape(_E))
        agg, st = _tc_b1(part, dvi)
        g2, b2, a2 = (g.reshape(1, _F) for g in gns[l])
        pw, pb = phis[l][0], phis[l][1].reshape(1, _R)
        h, pp, wmk = _tc_b2(agg, st, g2, b2, a2, nwc, ars[l + 1], pw, pb)
        wms.append(wmk)
        pps.append(pp)

    return _tc_final(wm0, wms[0], wms[1], wms[2], pps[0], pps[1], pps[2],
                     (r1_rho_w, r2_rho_w, r3_rho_w),
                     (r1_rho_b.reshape(1, _R), r2_rho_b.reshape(1, _R),
                      r3_rho_b.reshape(1, _R)))


# speedup vs baseline: 9.7578x; 1.0163x over previous
import jax
import jax.numpy as jnp
from jax import lax
from jax.experimental import pallas as pl
from jax.experimental.pallas import tpu as pltpu
from jax.experimental.pallas import tpu_sc as plsc

_N = 10000
_E = 320000
_F = 128
_R = 64
_NPAD = 10240
_NC = 2
_NS = 16
_NW = _NC * _NS
_ER = _E // 128
_RPW = _ER // _NW
_REM = _ER - _RPW * _NW
_WR = 2
_NWIN = _RPW // _WR
_BN = 2000


def _leaky(x):
    return jnp.where(x >= 0, x, 0.01 * x)


def _deg_body(src1, dst1, out_hbm, ho, hi, sbuf, dbuf):
    c = lax.axis_index("c")
    s = lax.axis_index("s")
    wid = c * _NS + s

    def _z(i, _):
        ho[pl.ds(i * 16, 16)] = jnp.zeros((16,), jnp.float32)
        hi[pl.ds(i * 16, 16)] = jnp.zeros((16,), jnp.float32)
        return 0

    lax.fori_loop(0, _NPAD // 16, _z, 0)

    base = (wid * _RPW + jnp.minimum(wid, _REM)) * 128
    ne = _RPW * 128
    pltpu.sync_copy(src1.at[pl.ds(base, ne)], sbuf.at[pl.ds(0, ne)])
    pltpu.sync_copy(dst1.at[pl.ds(base, ne)], dbuf.at[pl.ds(0, ne)])

    @pl.when(wid < _REM)
    def _():
        pltpu.sync_copy(src1.at[pl.ds(base + ne, 128)],
                        sbuf.at[pl.ds(ne, 128)])
        pltpu.sync_copy(dst1.at[pl.ds(base + ne, 128)],
                        dbuf.at[pl.ds(ne, 128)])

    ones = jnp.full((16,), 1.0, jnp.float32)

    def _sc(i, _):
        iv = sbuf[pl.ds(i * 16, 16)]
        plsc.addupdate_scatter(ho, [iv], ones)
        jv = dbuf[pl.ds(i * 16, 16)]
        plsc.addupdate_scatter(hi, [jv], ones)
        return 0

    lax.fori_loop(0, ne // 16, _sc, 0)

    @pl.when(wid < _REM)
    def _():
        lax.fori_loop(ne // 16, ne // 16 + 8, _sc, 0)

    pltpu.sync_copy(ho, out_hbm.at[pl.ds(wid * _NPAD, _NPAD)])
    pltpu.sync_copy(hi, out_hbm.at[pl.ds((_NW + wid) * _NPAD, _NPAD)])


def _prop_body(h_hbm, src1, dst1, ew1, out_hbm,
               acc, sbufs, dbufs, ewbs, rows, semi, semg, semsc):
    c = lax.axis_index("c")
    s = lax.axis_index("s")

    def _z(i, _):
        r = i // 8
        k = i % 8
        rows[0][r, pl.ds(k * 16, 16)] = jnp.zeros((16,), jnp.float32)
        return 0

    lax.fori_loop(0, 128 * 8, _z, 0)

    def _za(bk, _):
        pltpu.sync_copy(rows[0], acc.at[pl.ds(s * 624 + bk * 128, 128)])
        return 0

    lax.fori_loop(0, 4, _za, 0)
    pltpu.sync_copy(rows[0].at[pl.ds(0, 112)],
                    acc.at[pl.ds(s * 624 + 512, 112)])

    @pl.when(s == _NS - 1)
    def _():
        pltpu.sync_copy(rows[0].at[pl.ds(0, 16)], acc.at[pl.ds(9984, 16)])

    plsc.subcore_barrier()

    base_g = c * 1250 + s * 78 + jnp.minimum(s, 2)

    def _e0(w):
        return (base_g + w) * 128

    def _prefetch(w, m):
        e0 = _e0(w)
        pltpu.async_copy(src1.at[pl.ds(e0, 128)], sbufs[m], semi[m])
        pltpu.async_copy(dst1.at[pl.ds(e0, 128)], dbufs[m], semi[m])
        pltpu.async_copy(ew1.at[pl.ds(e0, 128)], ewbs[m], semi[m])

    def _wait_prefetch(w, m):
        e0 = _e0(w)
        pltpu.make_async_copy(src1.at[pl.ds(e0, 128)], sbufs[m], semi[m]).wait()
        pltpu.make_async_copy(dst1.at[pl.ds(e0, 128)], dbufs[m], semi[m]).wait()
        pltpu.make_async_copy(ew1.at[pl.ds(e0, 128)], ewbs[m], semi[m]).wait()

    def _gather(m, b):
        pltpu.async_copy(h_hbm.at[sbufs[m]], rows[b], semg[b])

    def _wait_gather(m, b):
        pltpu.make_async_copy(h_hbm.at[sbufs[m]], rows[b], semg[b]).wait()

    def _scale(m, b):
        def _grp(g, _):
            wv = ewbs[m][pl.ds(g * 16, 16)]
            for q in range(16):
                w_ = wv[q]
                r = g * 16 + q
                for f in range(8):
                    sl = pl.ds(f * 16, 16)
                    rows[b][r, sl] = rows[b][r, sl] * w_
            return 0

        lax.fori_loop(0, 8, _grp, 0)

    def _scatter(m, b):
        pltpu.async_copy(rows[b], acc.at[dbufs[m]], semsc[b], add=True)

    def _wait_scatter(m, b):
        pltpu.make_async_copy(rows[b], acc.at[dbufs[m]], semsc[b]).wait()

    def _win(w, t, guard_first, la1, la2):
        b = t % 3
        m = t % 4
        _wait_gather(m, b)
        if guard_first:
            @pl.when(w >= 1)
            def _():
                _wait_scatter((t - 1) % 4, (t - 1) % 3)
        else:
            _wait_scatter((t - 1) % 4, (t - 1) % 3)
        if la2:
            _wait_prefetch(w + 2, (t + 2) % 4)
            _gather((t + 2) % 4, (t + 2) % 3)
        if la1:
            _prefetch(w + 3, (t + 3) % 4)
        _scale(m, b)
        _scatter(m, b)

    _prefetch(0, 0)
    _prefetch(1, 1)
    _prefetch(2, 2)
    _wait_prefetch(0, 0)
    _gather(0, 0)
    _wait_prefetch(1, 1)
    _gather(1, 1)

    def _sb(k, _):
        w0 = k * 12
        for t in range(12):
            _win(w0 + t, t, guard_first=(t <= 1), la1=True, la2=True)
        return 0

    lax.fori_loop(0, 6, _sb, 0)

    for t in range(6):
        w = 72 + t
        _win(w, t, guard_first=False,
             la1=(w + 3 < 78), la2=(w + 2 < 78))
    _wait_scatter(77 % 4, 77 % 3)

    @pl.when(s < 2)
    def _():
        e0 = _e0(78)
        pltpu.sync_copy(src1.at[pl.ds(e0, 128)], sbufs[0])
        pltpu.sync_copy(dst1.at[pl.ds(e0, 128)], dbufs[0])
        pltpu.sync_copy(ew1.at[pl.ds(e0, 128)], ewbs[0])
        _gather(0, 0)
        _wait_gather(0, 0)
        _scale(0, 0)
        _scatter(0, 0)
        _wait_scatter(0, 0)

    plsc.subcore_barrier()
    pltpu.sync_copy(acc.at[pl.ds(s * 624, 624)],
                    out_hbm.at[c, pl.ds(s * 624, 624)])

    @pl.when(s == _NS - 1)
    def _():
        pltpu.sync_copy(acc.at[pl.ds(9984, 16)],
                        out_hbm.at[c, pl.ds(9984, 16)])


_sc_kernels = {}


def _get_sc_kernels():
    if "deg" not in _sc_kernels:
        mesh = plsc.VectorSubcoreMesh(
            core_axis_name="c", subcore_axis_name="s",
            num_cores=_NC, num_subcores=_NS)
        _sc_kernels["deg"] = pl.kernel(
            _deg_body,
            out_type=jax.ShapeDtypeStruct((2 * _NW * _NPAD,), jnp.float32),
            mesh=mesh,
            compiler_params=pltpu.CompilerParams(needs_layout_passes=False),
            scratch_types=[
                pltpu.VMEM((_NPAD,), jnp.float32),
                pltpu.VMEM((_NPAD,), jnp.float32),
                pltpu.VMEM(((_RPW + 1) * 128,), jnp.int32),
                pltpu.VMEM(((_RPW + 1) * 128,), jnp.int32),
            ],
        )
        _sc_kernels["prop"] = pl.kernel(
            _prop_body,
            out_type=jax.ShapeDtypeStruct((2, _N, _F), jnp.float32),
            mesh=mesh,
            scratch_types=[
                pltpu.VMEM_SHARED((_N, _F), jnp.float32),
                [pltpu.VMEM((128,), jnp.int32) for _ in range(4)],
                [pltpu.VMEM((128,), jnp.int32) for _ in range(4)],
                [pltpu.VMEM((128,), jnp.float32) for _ in range(4)],
                [pltpu.VMEM((128, 128), jnp.float32) for _ in range(3)],
                [pltpu.SemaphoreType.DMA for _ in range(4)],
                [pltpu.SemaphoreType.DMA for _ in range(3)],
                [pltpu.SemaphoreType.DMA for _ in range(3)],
            ],
        )
    return _sc_kernels


def _deg_call(srcr, dstr):
    return _get_sc_kernels()["deg"](srcr, dstr)


def _prop_call(hm, srcr, dstr, ewer):
    return _get_sc_kernels()["prop"](hm, srcr, dstr, ewer)


def _prep_body(h_ref, o_ref):
    x = h_ref[...]
    dego = jnp.sum(x[0:_NW], axis=0, keepdims=True)
    degi = jnp.sum(x[_NW:2 * _NW], axis=0, keepdims=True)
    deg = jnp.concatenate([dego, degi], axis=0)
    o_ref[...] = jax.lax.rsqrt(jnp.maximum(deg, 1.0))


def _a_body(x_ref, d_ref, nw_ref, ar_ref, w_ref, ew_ref, sp_ref,
            h_ref, ws_ref, ewe_ref):
    i = pl.program_id(0)
    x = x_ref[...]
    h_ref[...] = jnp.dot(x * d_ref[...], w_ref[...],
                         preferred_element_type=jnp.float32)
    ewe_ref[...] = ew_ref[...] * sp_ref[...]
    wm = jnp.sum(x * (nw_ref[...] * ar_ref[...]), axis=0, keepdims=True)

    @pl.when(i == 0)
    def _():
        ws_ref[...] = wm

    @pl.when(i != 0)
    def _():
        ws_ref[...] += wm


def _bba_body(p_ref, dvi_ref, g_ref, b_ref, al_ref, nw_ref, ar_ref,
              pw_ref, pb_ref, dvo_ref, w_ref, ew_ref, sp_ref,
              hm_ref, pp_ref, wm_ref, ewe_ref, st_ref):
    p = pl.program_id(0)
    i = pl.program_id(1)
    pr = p_ref[...]
    sgg = (pr[0] + pr[1]) * dvi_ref[...]
    ewe_ref[...] = ew_ref[...] * sp_ref[...]

    @pl.when(p == 0)
    def _():
        st = jnp.concatenate(
            [jnp.sum(sgg, axis=0, keepdims=True),
             jnp.sum(sgg * sgg, axis=0, keepdims=True)], axis=0)

        @pl.when(i == 0)
        def _():
            st_ref[...] = st

        @pl.when(i != 0)
        def _():
            st_ref[...] += st

    @pl.when(p == 1)
    def _():
        st = st_ref[...]
        m = st[0:1] / _N
        msq = st[1:2] / _N
        al = al_ref[...]
        var = msq - (2.0 * al - al * al) * (m * m)
        stdv = jnp.sqrt(var + 1e-5)
        xn = g_ref[...] * (sgg - al * m) / stdv + b_ref[...]
        hl = _leaky(xn)
        hm_ref[...] = jnp.dot(hl * dvo_ref[...], w_ref[...],
                              preferred_element_type=jnp.float32)
        phi = _leaky(jnp.dot(hl, pw_ref[...],
                             preferred_element_type=jnp.float32) + pb_ref[...])
        pp = jnp.sum(phi, axis=0, keepdims=True)
        wm = jnp.sum(hl * (nw_ref[...] * ar_ref[...]), axis=0, keepdims=True)

        @pl.when(i == 0)
        def _():
            pp_ref[...] = pp
            wm_ref[...] = wm

        @pl.when(i != 0)
        def _():
            pp_ref[...] += pp
            wm_ref[...] += wm


def _bbf_body(p_ref, dvi_ref, g_ref, b_ref, al_ref, nw_ref, ar_ref,
              pw_ref, pb_ref, wm0_ref, wm1_ref, wm2_ref, pp1_ref, pp2_ref,
              rw1_ref, rw2_ref, rw3_ref, rb1_ref, rb2_ref, rb3_ref,
              out_ref, st_ref, pp3_ref, wm3_ref):
    t = pl.program_id(0)
    g = _N // _BN
    pr = p_ref[...]
    sgg = (pr[0] + pr[1]) * dvi_ref[...]

    @pl.when(t < g)
    def _():
        st = jnp.concatenate(
            [jnp.sum(sgg, axis=0, keepdims=True),
             jnp.sum(sgg * sgg, axis=0, keepdims=True)], axis=0)

        @pl.when(t == 0)
        def _():
            st_ref[...] = st

        @pl.when(t != 0)
        def _():
            st_ref[...] += st

    @pl.when((t >= g) & (t < 2 * g))
    def _():
        st = st_ref[...]
        m = st[0:1] / _N
        msq = st[1:2] / _N
        al = al_ref[...]
        var = msq - (2.0 * al - al * al) * (m * m)
        stdv = jnp.sqrt(var + 1e-5)
        xn = g_ref[...] * (sgg - al * m) / stdv + b_ref[...]
        hl = _leaky(xn)
        phi = _leaky(jnp.dot(hl, pw_ref[...],
                             preferred_element_type=jnp.float32) + pb_ref[...])
        pp = jnp.sum(phi, axis=0, keepdims=True)
        wm = jnp.sum(hl * (nw_ref[...] * ar_ref[...]), axis=0, keepdims=True)

        @pl.when(t == g)
        def _():
            pp3_ref[...] = pp
            wm3_ref[...] = wm

        @pl.when(t != g)
        def _():
            pp3_ref[...] += pp
            wm3_ref[...] += wm

    @pl.when(t == 2 * g)
    def _():
        rs = []
        for pp, rw, rb in ((pp1_ref[...], rw1_ref, rb1_ref),
                           (pp2_ref[...], rw2_ref, rb2_ref),
                           (pp3_ref[...], rw3_ref, rb3_ref)):
            rs.append(_leaky(jnp.dot(pp / _N, rw[...],
                                     preferred_element_type=jnp.float32)
                             + rb[...]))
        out_ref[...] = _leaky(jnp.concatenate(
            [wm0_ref[...] / _N, rs[0], wm1_ref[...] / _N, rs[1],
             wm2_ref[...] / _N, rs[2], wm3_ref[...] / _N], axis=1))


def _tc_prep(hists):
    return pl.pallas_call(
        _prep_body,
        out_shape=jax.ShapeDtypeStruct((2, _NPAD), jnp.float32),
    )(hists)


def _tc_a(x, dvo, nwc, arc, w, ew2, sp2):
    g = _N // _BN
    eb = _E // g
    return pl.pallas_call(
        _a_body,
        grid=(g,),
        in_specs=[
            pl.BlockSpec((_BN, _F), lambda i: (i, 0)),
            pl.BlockSpec((_BN, 1), lambda i: (i, 0)),
            pl.BlockSpec((_BN, 1), lambda i: (i, 0)),
            pl.BlockSpec((_BN, 1), lambda i: (i, 0)),
            pl.BlockSpec((_F, _F), lambda i: (0, 0)),
            pl.BlockSpec((1, 1, eb), lambda i: (i, 0, 0)),
            pl.BlockSpec((1, 1, eb), lambda i: (i, 0, 0)),
        ],
        out_specs=[
            pl.BlockSpec((_BN, _F), lambda i: (i, 0)),
            pl.BlockSpec((1, _F), lambda i: (0, 0)),
            pl.BlockSpec((1, 1, eb), lambda i: (i, 0, 0)),
        ],
        out_shape=[
            jax.ShapeDtypeStruct((_N, _F), jnp.float32),
            jax.ShapeDtypeStruct((1, _F), jnp.float32),
            jax.ShapeDtypeStruct((g, 1, eb), jnp.float32),
        ],
    )(x, dvo, nwc, arc, w, ew2, sp2)


def _tc_bba(part, dvi, g2, b2, a2, nwc, arc, pw, pb, dvo, w, ew2, sp2):
    g = _N // _BN
    eb = _E // g
    return pl.pallas_call(
        _bba_body,
        grid=(2, g),
        in_specs=[
            pl.BlockSpec((2, _BN, _F), lambda p, i: (0, i, 0)),
            pl.BlockSpec((_BN, 1), lambda p, i: (i, 0)),
            pl.BlockSpec((1, _F), lambda p, i: (0, 0)),
            pl.BlockSpec((1, _F), lambda p, i: (0, 0)),
            pl.BlockSpec((1, _F), lambda p, i: (0, 0)),
            pl.BlockSpec((_BN, 1), lambda p, i: (i, 0)),
            pl.BlockSpec((_BN, 1), lambda p, i: (i, 0)),
            pl.BlockSpec((_F, _R), lambda p, i: (0, 0)),
            pl.BlockSpec((1, _R), lambda p, i: (0, 0)),
            pl.BlockSpec((_BN, 1), lambda p, i: (i, 0)),
            pl.BlockSpec((_F, _F), lambda p, i: (0, 0)),
            pl.BlockSpec((1, 1, eb), lambda p, i: (i, 0, 0)),
            pl.BlockSpec((1, 1, eb), lambda p, i: (i, 0, 0)),
        ],
        out_specs=[
            pl.BlockSpec((_BN, _F), lambda p, i: (i, 0)),
            pl.BlockSpec((1, _R), lambda p, i: (0, 0)),
            pl.BlockSpec((1, _F), lambda p, i: (0, 0)),
            pl.BlockSpec((1, 1, eb), lambda p, i: (i, 0, 0)),
        ],
        out_shape=[
            jax.ShapeDtypeStruct((_N, _F), jnp.float32),
            jax.ShapeDtypeStruct((1, _R), jnp.float32),
            jax.ShapeDtypeStruct((1, _F), jnp.float32),
            jax.ShapeDtypeStruct((g, 1, eb), jnp.float32),
        ],
        scratch_shapes=[pltpu.VMEM((2, _F), jnp.float32)],
    )(part, dvi, g2, b2, a2, nwc, arc, pw, pb, dvo, w, ew2, sp2)


def _tc_bbf(part, dvi, g2, b2, a2, nwc, arc, pw, pb,
            wm0, wm1, wm2, pp1, pp2, rws, rbs):
    g = _N // _BN

    def _pi(t):
        return jnp.where(t < g, t, jnp.where(t < 2 * g, t - g, 0))

    return pl.pallas_call(
        _bbf_body,
        grid=(2 * g + 1,),
        in_specs=[
            pl.BlockSpec((2, _BN, _F), lambda t: (0, _pi(t), 0)),
            pl.BlockSpec((_BN, 1), lambda t: (_pi(t), 0)),
            pl.BlockSpec((1, _F), lambda t: (0, 0)),
            pl.BlockSpec((1, _F), lambda t: (0, 0)),
            pl.BlockSpec((1, _F), lambda t: (0, 0)),
            pl.BlockSpec((_BN, 1), lambda t: (_pi(t), 0)),
            pl.BlockSpec((_BN, 1), lambda t: (_pi(t), 0)),
            pl.BlockSpec((_F, _R), lambda t: (0, 0)),
            pl.BlockSpec((1, _R), lambda t: (0, 0)),
            pl.BlockSpec((1, _F), lambda t: (0, 0)),
            pl.BlockSpec((1, _F), lambda t: (0, 0)),
            pl.BlockSpec((1, _F), lambda t: (0, 0)),
            pl.BlockSpec((1, _R), lambda t: (0, 0)),
            pl.BlockSpec((1, _R), lambda t: (0, 0)),
            pl.BlockSpec((_R, _R), lambda t: (0, 0)),
            pl.BlockSpec((_R, _R), lambda t: (0, 0)),
            pl.BlockSpec((_R, _R), lambda t: (0, 0)),
            pl.BlockSpec((1, _R), lambda t: (0, 0)),
            pl.BlockSpec((1, _R), lambda t: (0, 0)),
            pl.BlockSpec((1, _R), lambda t: (0, 0)),
        ],
        out_specs=[
            pl.BlockSpec((1, 704), lambda t: (0, 0)),
        ],
        out_shape=[
            jax.ShapeDtypeStruct((1, 704), jnp.float32),
        ],
        scratch_shapes=[
            pltpu.VMEM((2, _F), jnp.float32),
            pltpu.VMEM((1, _R), jnp.float32),
            pltpu.VMEM((1, _F), jnp.float32),
        ],
    )(part, dvi, g2, b2, a2, nwc, arc, pw, pb,
      wm0, wm1, wm2, pp1, pp2, rws[0], rws[1], rws[2],
      rbs[0], rbs[1], rbs[2])[0]


def kernel(node_feats, edge_weights, node_weights, suppl1, suppl2, suppl3,
           AR1, AR2, AR3, AR4, W1, W2, W3,
           gn1_gamma, gn1_beta, gn1_alpha,
           gn2_gamma, gn2_beta, gn2_alpha,
           gn3_gamma, gn3_beta, gn3_alpha,
           r1_phi_w, r1_phi_b, r1_rho_w, r1_rho_b,
           r2_phi_w, r2_phi_b, r2_rho_w, r2_rho_b,
           r3_phi_w, r3_phi_b, r3_rho_w, r3_rho_b, edge_index):
    srcr = edge_index[0]
    dstr = edge_index[1]
    g = _N // _BN
    eb = _E // g
    ew2 = edge_weights.reshape(g, 1, eb)
    sp2 = (suppl1.reshape(g, 1, eb), suppl2.reshape(g, 1, eb),
           suppl3.reshape(g, 1, eb))

    hists = _deg_call(srcr, dstr).reshape(2 * _NW, _NPAD)
    dinv2 = _tc_prep(hists)
    dvo = dinv2[0, :_N].reshape(_N, 1)
    dvi = dinv2[1, :_N].reshape(_N, 1)
    nwc = node_weights.reshape(_N, 1)
    ars = (AR1.reshape(_N, 1), AR2.reshape(_N, 1),
           AR3.reshape(_N, 1), AR4.reshape(_N, 1))
    gns = ((gn1_gamma, gn1_beta, gn1_alpha),
           (gn2_gamma, gn2_beta, gn2_alpha),
           (gn3_gamma, gn3_beta, gn3_alpha))
    phis = ((r1_phi_w, r1_phi_b), (r2_phi_w, r2_phi_b), (r3_phi_w, r3_phi_b))
    rws = (r1_rho_w, r2_rho_w, r3_rho_w)
    rbs = (r1_rho_b.reshape(1, _R), r2_rho_b.reshape(1, _R),
           r3_rho_b.reshape(1, _R))

    hm1, wm0, ewe1 = _tc_a(node_feats, dvo, nwc, ars[0], W1, ew2, sp2[0])
    part = _prop_call(hm1, srcr, dstr, ewe1.reshape(_E))

    wms, pps = [], []
    for l in range(2):
        g2, b2, a2 = (x.reshape(1, _F) for x in gns[l])
        pw, pb = phis[l][0], phis[l][1].reshape(1, _R)
        hm, pp, wmk, ewe = _tc_bba(part, dvi, g2, b2, a2, nwc, ars[l + 1],
                                   pw, pb, dvo, (W2, W3)[l], ew2, sp2[l + 1])
        wms.append(wmk)
        pps.append(pp)
        part = _prop_call(hm, srcr, dstr, ewe.reshape(_E))

    g2, b2, a2 = (x.reshape(1, _F) for x in gns[2])
    pw, pb = phis[2][0], phis[2][1].reshape(1, _R)
    return _tc_bbf(part, dvi, g2, b2, a2, nwc, ars[3], pw, pb,
                   wm0, wms[0], wms[1], pps[0], pps[1], rws, rbs)
